# Initial kernel scaffold; baseline (speedup 1.0000x reference)
import jax
import jax.numpy as jnp
from jax.experimental import pallas as pl


def kernel(x, W_enc, b_enc, W_dec, b_dec):
    raise NotImplementedError("write your pallas kernel here")


# speedup vs baseline: 7.4666x; 7.4666x over previous
import functools

import jax
import jax.numpy as jnp
from jax import lax
from jax.experimental import pallas as pl
from jax.experimental.pallas import tpu as pltpu

_IN = 2048
_HID = 8192
_NTOK = 8192
_K = 64
_INV_TAU = 2.0
_BT = 256
_BH = 1024


def _monotone_u32(f):
    b = lax.bitcast_convert_type(f, jnp.uint32)
    neg = b >= jnp.uint32(0x80000000)
    return jnp.where(neg, ~b, b | jnp.uint32(0x80000000))


def _make_body(bt, bh, in_dim, hid, k):
    hb = hid // bh

    def body(x_ref, we_ref, be_ref, wd_ref, bd_ref, cpt_ref, rec_ref,
             lg_scr, thr_scr, acc_scr):
        p = pl.program_id(1)

        @pl.when(p < hb)
        def _encode():
            lg = lax.dot_general(x_ref[...], we_ref[...],
                                 (((1,), (1,)), ((), ())),
                                 preferred_element_type=jnp.float32)
            lg = lg + be_ref[0, 0, :][None, :]
            lg_scr[:, pl.ds(pl.multiple_of(p * bh, bh), bh)] = lg

        @pl.when(p >= hb)
        def _decode():
            h = p - hb

            @pl.when(h == 0)
            def _threshold():
                def step(i, t):
                    sh = jnp.uint32(31) - i.astype(jnp.uint32)
                    trial = t | (jnp.uint32(1) << sh)

                    def csum(c, acc):
                        lg_c = lg_scr[:, pl.ds(pl.multiple_of(c * bh, bh), bh)]
                        ku_c = _monotone_u32(lg_c)
                        return acc + jnp.sum((ku_c >= trial).astype(jnp.int32),
                                             axis=1, keepdims=True)

                    cnt = lax.fori_loop(0, hb, csum,
                                        jnp.zeros((bt, 1), jnp.int32))
                    return jnp.where(cnt >= k, trial, t)

                t = lax.fori_loop(0, 32, step, jnp.zeros((bt, 1), jnp.uint32))
                thr_scr[...] = jnp.broadcast_to(t, (bt, 128))

            lg = lg_scr[:, pl.ds(pl.multiple_of(h * bh, bh), bh)]
            m = _monotone_u32(lg) >= thr_scr[:, 0:1]
            c = jnp.where(m, 1.0 / (1.0 + jnp.exp(lg * (-_INV_TAU))), 0.0)
            cpt_ref[...] = c
            contrib = lax.dot_general(c, wd_ref[...], (((1,), (1,)), ((), ())),
                                      preferred_element_type=jnp.float32)

            @pl.when(h == 0)
            def _():
                acc_scr[...] = contrib

            @pl.when(h > 0)
            def _():
                acc_scr[...] = acc_scr[...] + contrib

            @pl.when(h == hb - 1)
            def _():
                rec_ref[...] = acc_scr[...] + bd_ref[0, :][None, :]

    return body


@functools.partial(jax.jit, static_argnames=("bt", "bh", "n_tok", "in_dim",
                                             "hid", "k", "interpret"))
def _run(xb, web, be3, wdb, bd2, *, bt=_BT, bh=_BH, n_tok=_NTOK,
         in_dim=_IN, hid=_HID, k=_K, interpret=False):
    hb = hid // bh
    nt = n_tok // bt
    cpt, rec = pl.pallas_call(
        _make_body(bt, bh, in_dim, hid, k),
        grid=(nt, 2 * hb),
        in_specs=[
            pl.BlockSpec((bt, in_dim), lambda t, p: (t, 0)),
            pl.BlockSpec((bh, in_dim),
                         lambda t, p: (jnp.minimum(p, hb - 1), 0)),
            pl.BlockSpec((1, 1, bh),
                         lambda t, p: (jnp.minimum(p, hb - 1), 0, 0)),
            pl.BlockSpec((in_dim, bh),
                         lambda t, p: (0, jnp.maximum(p - hb, 0))),
            pl.BlockSpec((1, in_dim), lambda t, p: (0, 0)),
        ],
        out_specs=[
            pl.BlockSpec((bt, bh), lambda t, p: (t, jnp.maximum(p - hb, 0))),
            pl.BlockSpec((bt, in_dim), lambda t, p: (t, 0)),
        ],
        out_shape=[
            jax.ShapeDtypeStruct((n_tok, hid), jnp.float32),
            jax.ShapeDtypeStruct((n_tok, in_dim), jnp.float32),
        ],
        scratch_shapes=[
            pltpu.VMEM((bt, hid), jnp.float32),
            pltpu.VMEM((bt, 128), jnp.uint32),
            pltpu.VMEM((bt, in_dim), jnp.float32),
        ],
        compiler_params=pltpu.CompilerParams(
            dimension_semantics=("arbitrary", "arbitrary"),
        ),
        interpret=interpret,
    )(xb, web, be3, wdb, bd2)
    return rec, cpt


def kernel(x, W_enc, b_enc, W_dec, b_dec):
    xb = x.astype(jnp.bfloat16)
    web = W_enc.astype(jnp.bfloat16)
    wdb = W_dec.astype(jnp.bfloat16)
    hb = _HID // _BH
    be3 = b_enc.reshape(hb, 1, _BH)
    bd2 = b_dec.reshape(1, _IN)
    rec, cpt = _run(xb, web, be3, wdb, bd2)
    return (rec, cpt)


# speedup vs baseline: 17.0731x; 2.2866x over previous
import functools

import jax
import jax.numpy as jnp
from jax import lax
from jax.experimental import pallas as pl
from jax.experimental.pallas import tpu as pltpu

_IN = 2048
_HID = 8192
_NTOK = 8192
_K = 64
_INV_TAU = 2.0
_BT = 256
_BH = 1024
_NCAND = 8
_LANES = 128


_IMIN = -(2 ** 31)


def _monotone_i32(f):
    b = lax.bitcast_convert_type(f, jnp.int32)
    return jnp.where(b < 0, b ^ jnp.int32(0x7FFFFFFF), b)


def _search(ks_of_chunk, nchunk, bt, k):
    def step(i, t):
        trial = t + (jnp.int32(1) << (jnp.int32(31) - i.astype(jnp.int32)))

        def csum(c, acc):
            ks_c = ks_of_chunk(c)
            return acc + jnp.sum((ks_c >= trial).astype(jnp.int32),
                                 axis=1, keepdims=True)

        cnt = lax.fori_loop(0, nchunk, csum, jnp.zeros((bt, 1), jnp.int32))
        return jnp.where(cnt >= k, trial, t)

    return lax.fori_loop(0, 32, step, jnp.full((bt, 1), _IMIN, jnp.int32))


def _make_body(bt, bh, in_dim, hid, k):
    hb = hid // bh
    ngrp = bh // _LANES

    def body(x_ref, we_ref, be_ref, wd_ref, bd_ref, cpt_ref, rec_ref,
             lg_scr, thr_scr, acc_scr, cand_scr):
        p = pl.program_id(1)

        @pl.when(p < hb)
        def _encode():
            lg = lax.dot_general(x_ref[...], we_ref[...],
                                 (((1,), (1,)), ((), ())),
                                 preferred_element_type=jnp.float32)
            lg = lg + be_ref[0, 0, :][None, :]
            lg_scr[:, pl.ds(pl.multiple_of(p * bh, bh), bh)] = lg

            @pl.when(p == 0)
            def _():
                cand_scr[...] = jnp.full((bt, _NCAND * _LANES), _IMIN,
                                         jnp.int32)

            regs = [cand_scr[:, i * _LANES:(i + 1) * _LANES]
                    for i in range(_NCAND)]
            for g in range(ngrp):
                v = _monotone_i32(lg[:, g * _LANES:(g + 1) * _LANES])
                for i in range(_NCAND):
                    hi = jnp.maximum(regs[i], v)
                    v = jnp.minimum(regs[i], v)
                    regs[i] = hi
            for i in range(_NCAND):
                cand_scr[:, i * _LANES:(i + 1) * _LANES] = regs[i]

        @pl.when(p >= hb)
        def _decode():
            h = p - hb

            @pl.when(h == 0)
            def _threshold():
                cand = cand_scr[...]
                t = _search(lambda c: cand, 1, bt, k)

                def _full_chunk(c):
                    lg_c = lg_scr[:, pl.ds(pl.multiple_of(c * bh, bh), bh)]
                    return _monotone_i32(lg_c)

                def vsum(c, accs):
                    ge, gt = accs
                    ku_c = _full_chunk(c)
                    ge = ge + jnp.sum((ku_c >= t).astype(jnp.int32),
                                      axis=1, keepdims=True)
                    gt = gt + jnp.sum((ku_c > t).astype(jnp.int32),
                                      axis=1, keepdims=True)
                    return (ge, gt)

                zero = jnp.zeros((bt, 1), jnp.int32)
                ge, gt = lax.fori_loop(0, hb, vsum, (zero, zero))
                bad = jnp.logical_or(ge < k, gt >= k)
                thr_scr[...] = jnp.broadcast_to(t, (bt, 128))

                @pl.when(jnp.any(bad))
                def _fallback():
                    t_fb = _search(_full_chunk, hb, bt, k)
                    thr_scr[...] = jnp.where(
                        jnp.broadcast_to(bad, (bt, 128)),
                        jnp.broadcast_to(t_fb, (bt, 128)), thr_scr[...])

            lg = lg_scr[:, pl.ds(pl.multiple_of(h * bh, bh), bh)]
            m = _monotone_i32(lg) >= thr_scr[:, 0:1]
            c = jnp.where(m, 1.0 / (1.0 + jnp.exp(lg * (-_INV_TAU))), 0.0)
            cpt_ref[...] = c
            contrib = lax.dot_general(c, wd_ref[...], (((1,), (1,)), ((), ())),
                                      preferred_element_type=jnp.float32)

            @pl.when(h == 0)
            def _():
                acc_scr[...] = contrib

            @pl.when(h > 0)
            def _():
                acc_scr[...] = acc_scr[...] + contrib

            @pl.when(h == hb - 1)
            def _():
                rec_ref[...] = acc_scr[...] + bd_ref[0, :][None, :]

    return body


@functools.partial(jax.jit, static_argnames=("bt", "bh", "n_tok", "in_dim",
                                             "hid", "k", "interpret"))
def _run(xb, web, be3, wdb, bd2, *, bt=_BT, bh=_BH, n_tok=_NTOK,
         in_dim=_IN, hid=_HID, k=_K, interpret=False):
    hb = hid // bh
    nt = n_tok // bt
    cpt, rec = pl.pallas_call(
        _make_body(bt, bh, in_dim, hid, k),
        grid=(nt, 2 * hb),
        in_specs=[
            pl.BlockSpec((bt, in_dim), lambda t, p: (t, 0)),
            pl.BlockSpec((bh, in_dim),
                         lambda t, p: (jnp.minimum(p, hb - 1), 0)),
            pl.BlockSpec((1, 1, bh),
                         lambda t, p: (jnp.minimum(p, hb - 1), 0, 0)),
            pl.BlockSpec((in_dim, bh),
                         lambda t, p: (0, jnp.maximum(p - hb, 0))),
            pl.BlockSpec((1, in_dim), lambda t, p: (0, 0)),
        ],
        out_specs=[
            pl.BlockSpec((bt, bh), lambda t, p: (t, jnp.maximum(p - hb, 0))),
            pl.BlockSpec((bt, in_dim), lambda t, p: (t, 0)),
        ],
        out_shape=[
            jax.ShapeDtypeStruct((n_tok, hid), jnp.float32),
            jax.ShapeDtypeStruct((n_tok, in_dim), jnp.float32),
        ],
        scratch_shapes=[
            pltpu.VMEM((bt, hid), jnp.float32),
            pltpu.VMEM((bt, 128), jnp.int32),
            pltpu.VMEM((bt, in_dim), jnp.float32),
            pltpu.VMEM((bt, _NCAND * _LANES), jnp.int32),
        ],
        compiler_params=pltpu.CompilerParams(
            dimension_semantics=("arbitrary", "arbitrary"),
        ),
        interpret=interpret,
    )(xb, web, be3, wdb, bd2)
    return rec, cpt


def kernel(x, W_enc, b_enc, W_dec, b_dec):
    xb = x.astype(jnp.bfloat16)
    web = W_enc.astype(jnp.bfloat16)
    wdb = W_dec.astype(jnp.bfloat16)
    hb = _HID // _BH
    be3 = b_enc.reshape(hb, 1, _BH)
    bd2 = b_dec.reshape(1, _IN)
    rec, cpt = _run(xb, web, be3, wdb, bd2)
    return (rec, cpt)


# speedup vs baseline: 18.5290x; 1.0853x over previous
import functools

import jax
import jax.numpy as jnp
from jax import lax
from jax.experimental import pallas as pl
from jax.experimental.pallas import tpu as pltpu

_IN = 2048
_HID = 8192
_NTOK = 8192
_K = 64
_INV_TAU = 2.0
_BT = 256
_BH = 1024
_NCAND = 8
_LANES = 128


_IMIN = -(2 ** 31)


def _monotone_i32(f):
    b = lax.bitcast_convert_type(f, jnp.int32)
    return jnp.where(b < 0, b ^ jnp.int32(0x7FFFFFFF), b)


def _key_to_f32(t):
    b = jnp.where(t < 0, t ^ jnp.int32(0x7FFFFFFF), t)
    return lax.bitcast_convert_type(b, jnp.float32)


def _search(ks_of_chunk, nchunk, bt, k):
    def step(i, t):
        trial = t + (jnp.int32(1) << (jnp.int32(31) - i.astype(jnp.int32)))

        def csum(c, acc):
            ks_c = ks_of_chunk(c)
            return acc + jnp.sum((ks_c >= trial).astype(jnp.int32),
                                 axis=1, keepdims=True)

        cnt = lax.fori_loop(0, nchunk, csum, jnp.zeros((bt, 1), jnp.int32))
        return jnp.where(cnt >= k, trial, t)

    return lax.fori_loop(0, 32, step, jnp.full((bt, 1), _IMIN, jnp.int32))


def _make_body(bt, bh, in_dim, hid, k):
    hb = hid // bh
    ngrp = bh // _LANES

    def body(x_ref, we_ref, be_ref, wd_ref, bd_ref, cpt_ref, rec_ref,
             lg_scr, thr_scr, acc_scr, cand_scr):
        p = pl.program_id(1)

        @pl.when(p < hb)
        def _encode():
            lg = lax.dot_general(x_ref[...], we_ref[...],
                                 (((1,), (1,)), ((), ())),
                                 preferred_element_type=jnp.float32)
            lg = lg + be_ref[0, 0, :][None, :]
            lg_scr[:, pl.ds(pl.multiple_of(p * bh, bh), bh)] = lg

            @pl.when(p == 0)
            def _():
                cand_scr[...] = jnp.full((bt, _NCAND * _LANES), -jnp.inf,
                                         jnp.float32)

            regs = [cand_scr[:, i * _LANES:(i + 1) * _LANES]
                    for i in range(_NCAND)]
            for g in range(ngrp):
                v = lg[:, g * _LANES:(g + 1) * _LANES]
                for i in range(_NCAND):
                    hi = jnp.maximum(regs[i], v)
                    v = jnp.minimum(regs[i], v)
                    regs[i] = hi
            for i in range(_NCAND):
                cand_scr[:, i * _LANES:(i + 1) * _LANES] = regs[i]

        @pl.when(p >= hb)
        def _decode():
            h = p - hb

            @pl.when(h == 0)
            def _threshold():
                ks_cand = _monotone_i32(cand_scr[...])
                t = _search(lambda c: ks_cand, 1, bt, k)
                thr = _key_to_f32(t)

                def vsum(c, accs):
                    ge, gt = accs
                    lg_c = lg_scr[:, pl.ds(pl.multiple_of(c * bh, bh), bh)]
                    ge = ge + jnp.sum((lg_c >= thr).astype(jnp.int32),
                                      axis=1, keepdims=True)
                    gt = gt + jnp.sum((lg_c > thr).astype(jnp.int32),
                                      axis=1, keepdims=True)
                    return (ge, gt)

                zero = jnp.zeros((bt, 1), jnp.int32)
                ge, gt = lax.fori_loop(0, hb, vsum, (zero, zero))
                bad = jnp.logical_or(ge < k, gt >= k)
                thr_scr[...] = jnp.broadcast_to(thr, (bt, 128))

                @pl.when(jnp.any(bad))
                def _fallback():
                    def _full_chunk(c):
                        lg_c = lg_scr[:, pl.ds(pl.multiple_of(c * bh, bh),
                                               bh)]
                        return _monotone_i32(lg_c)

                    t_fb = _key_to_f32(_search(_full_chunk, hb, bt, k))
                    thr_scr[...] = jnp.where(
                        jnp.broadcast_to(bad, (bt, 128)),
                        jnp.broadcast_to(t_fb, (bt, 128)), thr_scr[...])

            lg = lg_scr[:, pl.ds(pl.multiple_of(h * bh, bh), bh)]
            m = lg >= thr_scr[:, 0:1]
            c = jnp.where(m, 0.5 + 0.5 * jnp.tanh(lg), 0.0)
            cpt_ref[...] = c
            contrib = lax.dot_general(c, wd_ref[...], (((1,), (1,)), ((), ())),
                                      preferred_element_type=jnp.float32)

            @pl.when(h == 0)
            def _():
                acc_scr[...] = contrib

            @pl.when(h > 0)
            def _():
                acc_scr[...] = acc_scr[...] + contrib

            @pl.when(h == hb - 1)
            def _():
                rec_ref[...] = acc_scr[...] + bd_ref[0, :][None, :]

    return body


@functools.partial(jax.jit, static_argnames=("bt", "bh", "n_tok", "in_dim",
                                             "hid", "k", "interpret"))
def _run(xb, web, be3, wdb, bd2, *, bt=_BT, bh=_BH, n_tok=_NTOK,
         in_dim=_IN, hid=_HID, k=_K, interpret=False):
    hb = hid // bh
    nt = n_tok // bt
    cpt, rec = pl.pallas_call(
        _make_body(bt, bh, in_dim, hid, k),
        grid=(nt, 2 * hb),
        in_specs=[
            pl.BlockSpec((bt, in_dim), lambda t, p: (t, 0)),
            pl.BlockSpec((bh, in_dim),
                         lambda t, p: (jnp.minimum(p, hb - 1), 0)),
            pl.BlockSpec((1, 1, bh),
                         lambda t, p: (jnp.minimum(p, hb - 1), 0, 0)),
            pl.BlockSpec((in_dim, bh),
                         lambda t, p: (0, jnp.maximum(p - hb, 0))),
            pl.BlockSpec((1, in_dim), lambda t, p: (0, 0)),
        ],
        out_specs=[
            pl.BlockSpec((bt, bh), lambda t, p: (t, jnp.maximum(p - hb, 0))),
            pl.BlockSpec((bt, in_dim), lambda t, p: (t, 0)),
        ],
        out_shape=[
            jax.ShapeDtypeStruct((n_tok, hid), jnp.float32),
            jax.ShapeDtypeStruct((n_tok, in_dim), jnp.float32),
        ],
        scratch_shapes=[
            pltpu.VMEM((bt, hid), jnp.float32),
            pltpu.VMEM((bt, 128), jnp.float32),
            pltpu.VMEM((bt, in_dim), jnp.float32),
            pltpu.VMEM((bt, _NCAND * _LANES), jnp.float32),
        ],
        compiler_params=pltpu.CompilerParams(
            dimension_semantics=("arbitrary", "arbitrary"),
        ),
        interpret=interpret,
    )(xb, web, be3, wdb, bd2)
    return rec, cpt


def kernel(x, W_enc, b_enc, W_dec, b_dec):
    xb = x.astype(jnp.bfloat16)
    web = W_enc.astype(jnp.bfloat16)
    wdb = W_dec.astype(jnp.bfloat16)
    hb = _HID // _BH
    be3 = b_enc.reshape(hb, 1, _BH)
    bd2 = b_dec.reshape(1, _IN)
    rec, cpt = _run(xb, web, be3, wdb, bd2)
    return (rec, cpt)


# speedup vs baseline: 19.7250x; 1.0645x over previous
import functools

import jax
import jax.numpy as jnp
from jax import lax
from jax.experimental import pallas as pl
from jax.experimental.pallas import tpu as pltpu

_IN = 2048
_HID = 8192
_NTOK = 8192
_K = 64
_INV_TAU = 2.0
_BT = 256
_BH = 1024
_NCAND = 8
_LANES = 128


_IMIN = -(2 ** 31)


def _monotone_i32(f):
    b = lax.bitcast_convert_type(f, jnp.int32)
    return jnp.where(b < 0, b ^ jnp.int32(0x7FFFFFFF), b)


def _key_to_f32(t):
    b = jnp.where(t < 0, t ^ jnp.int32(0x7FFFFFFF), t)
    return lax.bitcast_convert_type(b, jnp.float32)


def _search(ks_of_chunk, nchunk, bt, k):
    def step(i, t):
        sh = jnp.int32(30) - 2 * i.astype(jnp.int32)
        q1 = jnp.int32(1) << (sh + 1)
        q0 = jnp.int32(1) << sh
        t11 = t + q1 + q0
        t10 = t + q1
        t01 = t + q0

        def csum(c, accs):
            a11, a10, a01 = accs
            ks_c = ks_of_chunk(c)
            a11 = a11 + jnp.sum((ks_c >= t11).astype(jnp.int32),
                                axis=1, keepdims=True)
            a10 = a10 + jnp.sum((ks_c >= t10).astype(jnp.int32),
                                axis=1, keepdims=True)
            a01 = a01 + jnp.sum((ks_c >= t01).astype(jnp.int32),
                                axis=1, keepdims=True)
            return (a11, a10, a01)

        zero = jnp.zeros((bt, 1), jnp.int32)
        c11, c10, c01 = lax.fori_loop(0, nchunk, csum, (zero, zero, zero))
        return jnp.where(c11 >= k, t11,
                         jnp.where(c10 >= k, t10,
                                   jnp.where(c01 >= k, t01, t)))

    return lax.fori_loop(0, 16, step, jnp.full((bt, 1), _IMIN, jnp.int32))


def _make_body(bt, bh, in_dim, hid, k):
    hb = hid // bh
    ngrp = bh // _LANES

    def body(x_ref, we_ref, be_ref, wd_ref, bd_ref, cpt_ref, rec_ref,
             lg_scr, thr_scr, acc_scr, cand_scr, disp_scr):
        p = pl.program_id(1)

        @pl.when(p < hb)
        def _encode():
            lg = lax.dot_general(x_ref[...], we_ref[...],
                                 (((1,), (1,)), ((), ())),
                                 preferred_element_type=jnp.float32)
            lg = lg + be_ref[0, 0, :][None, :]
            lg_scr[:, pl.ds(pl.multiple_of(p * bh, bh), bh)] = lg

            @pl.when(p == 0)
            def _():
                cand_scr[...] = jnp.full((bt, _NCAND * _LANES), -jnp.inf,
                                         jnp.float32)
                disp_scr[...] = jnp.full((bt, _LANES), -jnp.inf, jnp.float32)

            regs = [cand_scr[:, i * _LANES:(i + 1) * _LANES]
                    for i in range(_NCAND)]
            disp = disp_scr[...]
            for g in range(ngrp):
                v = lg[:, g * _LANES:(g + 1) * _LANES]
                for i in range(_NCAND):
                    hi = jnp.maximum(regs[i], v)
                    v = jnp.minimum(regs[i], v)
                    regs[i] = hi
                disp = jnp.maximum(disp, v)
            disp_scr[...] = disp
            for i in range(_NCAND):
                cand_scr[:, i * _LANES:(i + 1) * _LANES] = regs[i]

        @pl.when(p >= hb)
        def _decode():
            h = p - hb

            @pl.when(h == 0)
            def _threshold():
                ks_cand = _monotone_i32(cand_scr[...])
                t = _search(lambda c: ks_cand, 1, bt, k)
                thr = _key_to_f32(t)
                d_max = jnp.max(disp_scr[...], axis=1, keepdims=True)
                bad = jnp.logical_not(thr > d_max)
                thr_scr[...] = jnp.broadcast_to(thr, (bt, 128))

                @pl.when(jnp.any(bad))
                def _fallback():
                    def _full_chunk(c):
                        lg_c = lg_scr[:, pl.ds(pl.multiple_of(c * bh, bh),
                                               bh)]
                        return _monotone_i32(lg_c)

                    t_fb = _key_to_f32(_search(_full_chunk, hb, bt, k))
                    thr_scr[...] = jnp.where(
                        jnp.broadcast_to(bad, (bt, 128)),
                        jnp.broadcast_to(t_fb, (bt, 128)), thr_scr[...])

            lg = lg_scr[:, pl.ds(pl.multiple_of(h * bh, bh), bh)]
            m = lg >= thr_scr[:, 0:1]
            c = jnp.where(m, 0.5 + 0.5 * jnp.tanh(lg), 0.0)
            cpt_ref[...] = c
            contrib = lax.dot_general(c, wd_ref[...], (((1,), (1,)), ((), ())),
                                      preferred_element_type=jnp.float32)

            @pl.when(h == 0)
            def _():
                acc_scr[...] = contrib

            @pl.when(h > 0)
            def _():
                acc_scr[...] = acc_scr[...] + contrib

            @pl.when(h == hb - 1)
            def _():
                rec_ref[...] = acc_scr[...] + bd_ref[0, :][None, :]

    return body


@functools.partial(jax.jit, static_argnames=("bt", "bh", "n_tok", "in_dim",
                                             "hid", "k", "interpret"))
def _run(xb, web, be3, wdb, bd2, *, bt=_BT, bh=_BH, n_tok=_NTOK,
         in_dim=_IN, hid=_HID, k=_K, interpret=False):
    hb = hid // bh
    nt = n_tok // bt
    cpt, rec = pl.pallas_call(
        _make_body(bt, bh, in_dim, hid, k),
        grid=(nt, 2 * hb),
        in_specs=[
            pl.BlockSpec((bt, in_dim), lambda t, p: (t, 0)),
            pl.BlockSpec((bh, in_dim),
                         lambda t, p: (jnp.minimum(p, hb - 1), 0)),
            pl.BlockSpec((1, 1, bh),
                         lambda t, p: (jnp.minimum(p, hb - 1), 0, 0)),
            pl.BlockSpec((in_dim, bh),
                         lambda t, p: (0, jnp.maximum(p - hb, 0))),
            pl.BlockSpec((1, in_dim), lambda t, p: (0, 0)),
        ],
        out_specs=[
            pl.BlockSpec((bt, bh), lambda t, p: (t, jnp.maximum(p - hb, 0))),
            pl.BlockSpec((bt, in_dim), lambda t, p: (t, 0)),
        ],
        out_shape=[
            jax.ShapeDtypeStruct((n_tok, hid), jnp.float32),
            jax.ShapeDtypeStruct((n_tok, in_dim), jnp.float32),
        ],
        scratch_shapes=[
            pltpu.VMEM((bt, hid), jnp.float32),
            pltpu.VMEM((bt, 128), jnp.float32),
            pltpu.VMEM((bt, in_dim), jnp.float32),
            pltpu.VMEM((bt, _NCAND * _LANES), jnp.float32),
            pltpu.VMEM((bt, _LANES), jnp.float32),
        ],
        compiler_params=pltpu.CompilerParams(
            dimension_semantics=("arbitrary", "arbitrary"),
        ),
        interpret=interpret,
    )(xb, web, be3, wdb, bd2)
    return rec, cpt


def kernel(x, W_enc, b_enc, W_dec, b_dec):
    xb = x.astype(jnp.bfloat16)
    web = W_enc.astype(jnp.bfloat16)
    wdb = W_dec.astype(jnp.bfloat16)
    hb = _HID // _BH
    be3 = b_enc.reshape(hb, 1, _BH)
    bd2 = b_dec.reshape(1, _IN)
    rec, cpt = _run(xb, web, be3, wdb, bd2)
    return (rec, cpt)


# speedup vs baseline: 20.8442x; 1.0567x over previous
import functools

import jax
import jax.numpy as jnp
from jax import lax
from jax.experimental import pallas as pl
from jax.experimental.pallas import tpu as pltpu

_IN = 2048
_HID = 8192
_NTOK = 8192
_K = 64
_INV_TAU = 2.0
_BT = 512
_BH = 512
_NCAND = 8
_LANES = 128


_IMIN = -(2 ** 31)


def _monotone_i32(f):
    b = lax.bitcast_convert_type(f, jnp.int32)
    return jnp.where(b < 0, b ^ jnp.int32(0x7FFFFFFF), b)


def _key_to_f32(t):
    b = jnp.where(t < 0, t ^ jnp.int32(0x7FFFFFFF), t)
    return lax.bitcast_convert_type(b, jnp.float32)


def _search(ks_of_chunk, nchunk, bt, k):
    def step(i, t):
        sh = jnp.int32(30) - 2 * i.astype(jnp.int32)
        q1 = jnp.int32(1) << (sh + 1)
        q0 = jnp.int32(1) << sh
        t11 = t + q1 + q0
        t10 = t + q1
        t01 = t + q0

        def csum(c, accs):
            a11, a10, a01 = accs
            ks_c = ks_of_chunk(c)
            a11 = a11 + jnp.sum((ks_c >= t11).astype(jnp.int32),
                                axis=1, keepdims=True)
            a10 = a10 + jnp.sum((ks_c >= t10).astype(jnp.int32),
                                axis=1, keepdims=True)
            a01 = a01 + jnp.sum((ks_c >= t01).astype(jnp.int32),
                                axis=1, keepdims=True)
            return (a11, a10, a01)

        zero = jnp.zeros((bt, 1), jnp.int32)
        c11, c10, c01 = lax.fori_loop(0, nchunk, csum, (zero, zero, zero))
        return jnp.where(c11 >= k, t11,
                         jnp.where(c10 >= k, t10,
                                   jnp.where(c01 >= k, t01, t)))

    return lax.fori_loop(0, 16, step, jnp.full((bt, 1), _IMIN, jnp.int32))


def _make_body(bt, bh, in_dim, hid, k):
    hb = hid // bh
    ngrp = bh // _LANES

    def body(x_ref, we_ref, be_ref, wd_ref, bd_ref, cpt_ref, rec_ref,
             lg_scr, thr_scr, acc_scr, cand_scr, disp_scr):
        p = pl.program_id(1)

        @pl.when(p < hb)
        def _encode():
            lg = lax.dot_general(x_ref[...], we_ref[...],
                                 (((1,), (1,)), ((), ())),
                                 preferred_element_type=jnp.float32)
            lg = lg + be_ref[0, 0, :][None, :]
            lg_scr[:, pl.ds(pl.multiple_of(p * bh, bh), bh)] = lg

            @pl.when(p == 0)
            def _():
                cand_scr[...] = jnp.full((bt, _NCAND * _LANES), -jnp.inf,
                                         jnp.float32)
                disp_scr[...] = jnp.full((bt, _LANES), -jnp.inf, jnp.float32)

            regs = [cand_scr[:, i * _LANES:(i + 1) * _LANES]
                    for i in range(_NCAND)]
            disp = disp_scr[...]
            for g in range(ngrp):
                v = lg[:, g * _LANES:(g + 1) * _LANES]
                for i in range(_NCAND):
                    hi = jnp.maximum(regs[i], v)
                    v = jnp.minimum(regs[i], v)
                    regs[i] = hi
                disp = jnp.maximum(disp, v)
            disp_scr[...] = disp
            for i in range(_NCAND):
                cand_scr[:, i * _LANES:(i + 1) * _LANES] = regs[i]

        @pl.when(p >= hb)
        def _decode():
            h = p - hb

            @pl.when(h == 0)
            def _threshold():
                ks_cand = _monotone_i32(cand_scr[...])
                t = _search(lambda c: ks_cand, 1, bt, k)
                thr = _key_to_f32(t)
                d_max = jnp.max(disp_scr[...], axis=1, keepdims=True)
                bad = jnp.logical_not(thr > d_max)
                thr_scr[...] = jnp.broadcast_to(thr, (bt, 128))

                @pl.when(jnp.any(bad))
                def _fallback():
                    def _full_chunk(c):
                        lg_c = lg_scr[:, pl.ds(pl.multiple_of(c * bh, bh),
                                               bh)]
                        return _monotone_i32(lg_c)

                    t_fb = _key_to_f32(_search(_full_chunk, hb, bt, k))
                    thr_scr[...] = jnp.where(
                        jnp.broadcast_to(bad, (bt, 128)),
                        jnp.broadcast_to(t_fb, (bt, 128)), thr_scr[...])

            lg = lg_scr[:, pl.ds(pl.multiple_of(h * bh, bh), bh)]
            m = lg >= thr_scr[:, 0:1]
            c = jnp.where(m, 0.5 + 0.5 * jnp.tanh(lg), 0.0)
            cpt_ref[...] = c
            contrib = lax.dot_general(c, wd_ref[...], (((1,), (1,)), ((), ())),
                                      preferred_element_type=jnp.float32)

            @pl.when(h == 0)
            def _():
                acc_scr[...] = contrib

            @pl.when(h > 0)
            def _():
                acc_scr[...] = acc_scr[...] + contrib

            @pl.when(h == hb - 1)
            def _():
                rec_ref[...] = acc_scr[...] + bd_ref[0, :][None, :]

    return body


@functools.partial(jax.jit, static_argnames=("bt", "bh", "n_tok", "in_dim",
                                             "hid", "k", "interpret"))
def _run(xb, web, be3, wdb, bd2, *, bt=_BT, bh=_BH, n_tok=_NTOK,
         in_dim=_IN, hid=_HID, k=_K, interpret=False):
    hb = hid // bh
    nt = n_tok // bt
    cpt, rec = pl.pallas_call(
        _make_body(bt, bh, in_dim, hid, k),
        grid=(nt, 2 * hb),
        in_specs=[
            pl.BlockSpec((bt, in_dim), lambda t, p: (t, 0)),
            pl.BlockSpec((bh, in_dim),
                         lambda t, p: (jnp.minimum(p, hb - 1), 0)),
            pl.BlockSpec((1, 1, bh),
                         lambda t, p: (jnp.minimum(p, hb - 1), 0, 0)),
            pl.BlockSpec((in_dim, bh),
                         lambda t, p: (0, jnp.maximum(p - hb, 0))),
            pl.BlockSpec((1, in_dim), lambda t, p: (0, 0)),
        ],
        out_specs=[
            pl.BlockSpec((bt, bh), lambda t, p: (t, jnp.maximum(p - hb, 0))),
            pl.BlockSpec((bt, in_dim), lambda t, p: (t, 0)),
        ],
        out_shape=[
            jax.ShapeDtypeStruct((n_tok, hid), jnp.float32),
            jax.ShapeDtypeStruct((n_tok, in_dim), jnp.float32),
        ],
        scratch_shapes=[
            pltpu.VMEM((bt, hid), jnp.float32),
            pltpu.VMEM((bt, 128), jnp.float32),
            pltpu.VMEM((bt, in_dim), jnp.float32),
            pltpu.VMEM((bt, _NCAND * _LANES), jnp.float32),
            pltpu.VMEM((bt, _LANES), jnp.float32),
        ],
        compiler_params=pltpu.CompilerParams(
            dimension_semantics=("arbitrary", "arbitrary"),
        ),
        interpret=interpret,
    )(xb, web, be3, wdb, bd2)
    return rec, cpt


def kernel(x, W_enc, b_enc, W_dec, b_dec):
    xb = x.astype(jnp.bfloat16)
    web = W_enc.astype(jnp.bfloat16)
    wdb = W_dec.astype(jnp.bfloat16)
    hb = _HID // _BH
    be3 = b_enc.reshape(hb, 1, _BH)
    bd2 = b_dec.reshape(1, _IN)
    rec, cpt = _run(xb, web, be3, wdb, bd2)
    return (rec, cpt)
